# Initial kernel scaffold; baseline (speedup 1.0000x reference)
import jax
import jax.numpy as jnp
from jax.experimental import pallas as pl


def kernel(x, target):
    raise NotImplementedError("write your pallas kernel here")


# speedup vs baseline: 1.8690x; 1.8690x over previous
import math

import jax
import jax.numpy as jnp
from jax.experimental import pallas as pl
from jax.experimental.pallas import tpu as pltpu

_C = 100000
_SMOOTHING = 0.1
_CONF = 1.0 - _SMOOTHING
_FILL = _SMOOTHING / (_C - 1)
_CONST = (_C - 1) * _FILL * math.log(_FILL) + _CONF * math.log(_CONF)

_N = 1024
_BR = 128
_BC = 12800


def _body(t_ref, x_ref, o_ref):
    i = pl.program_id(0)
    j = pl.program_id(1)

    @pl.when((i == 0) & (j == 0))
    def _init():
        o_ref[...] = jnp.full((1, 1), _CONST, dtype=jnp.float32)

    blk = x_ref[...]
    col = jax.lax.broadcasted_iota(jnp.int32, (_BR, _BC), 1) + j * _BC
    xv = jnp.where(col < _C, blk, 0.0)
    s = jnp.sum(xv)
    t = t_ref[...]
    g = jnp.sum(jnp.where(col == t, xv, 0.0))
    upd = jnp.float32(-_FILL / _N) * s + jnp.float32(-(_CONF - _FILL) / _N) * g
    o_ref[...] += upd.reshape(1, 1)


def kernel(x, target):
    B, M, C = x.shape
    n = B * M
    x2 = x.reshape(n, C)
    t2 = target.reshape(n, 1).astype(jnp.int32)
    grid = (n // _BR, pl.cdiv(C, _BC))
    out = pl.pallas_call(
        _body,
        grid=grid,
        in_specs=[
            pl.BlockSpec((_BR, 1), lambda i, j: (i, 0)),
            pl.BlockSpec((_BR, _BC), lambda i, j: (i, j)),
        ],
        out_specs=pl.BlockSpec((1, 1), lambda i, j: (0, 0)),
        out_shape=jax.ShapeDtypeStruct((1, 1), jnp.float32),
    )(t2, x2)
    return out[0, 0]
